# baseline (speedup 1.0000x reference)
import jax
import jax.numpy as jnp
from jax.experimental import pallas as pl

N = 10000
G = 64


def _gcn_conv(x, src2, dst2, dinv, W, b):
    xw = x @ W
    norm = (dinv[src2] * dinv[dst2])[:, None]
    msg = jnp.take(xw, src2, axis=0) * norm
    out = jax.ops.segment_sum(msg, dst2, num_segments=N)
    return out + b


def _final_kernel(pooled_ref, w_ref, b_ref, o_ref):
    o_ref[...] = pooled_ref[...] @ w_ref[...] + b_ref[...]


def kernel(x, edge_index, batch, W1, b1, W2, b2, W3, b3):
    src, dst = edge_index[0], edge_index[1]
    loop = jnp.arange(N, dtype=src.dtype)
    src2 = jnp.concatenate([src, loop])
    dst2 = jnp.concatenate([dst, loop])
    deg = jax.ops.segment_sum(jnp.ones(src2.shape, dtype=x.dtype), dst2, num_segments=N)
    dinv = jax.lax.rsqrt(jnp.maximum(deg, 1.0))
    h = jax.nn.relu(_gcn_conv(x, src2, dst2, dinv, W1, b1))
    h = jax.nn.relu(_gcn_conv(h, src2, dst2, dinv, W2, b2))
    summed = jax.ops.segment_sum(h, batch, num_segments=G)
    counts = jax.ops.segment_sum(jnp.ones(batch.shape, dtype=h.dtype), batch, num_segments=G)
    pooled = summed / jnp.maximum(counts, 1.0)[:, None]
    out = pl.pallas_call(
        _final_kernel,
        out_shape=jax.ShapeDtypeStruct((G, W3.shape[1]), jnp.float32),
    )(pooled, W3, b3[None, :])
    return out


# speedup vs baseline: 14.6599x; 14.6599x over previous
import functools

import jax
import jax.numpy as jnp
from jax import lax
from jax.experimental import pallas as pl
from jax.experimental.pallas import tpu as pltpu
from jax.experimental.pallas import tpu_sc as plsc

N = 10000
E = 320000
D = 128
H = 64
C = 32
G = 64

NC = 2
NS = 16
NW = NC * NS
NPAD = 10240
RPS = NPAD // NS
EW = E // NW
CH = 80
NCH = EW // CH

R = 2048
NB = NPAD // R

_mesh = plsc.VectorSubcoreMesh(core_axis_name="c", subcore_axis_name="s")


@functools.partial(
    pl.kernel,
    out_type=jax.ShapeDtypeStruct((NC, NPAD), jnp.float32),
    mesh=_mesh,
    scratch_types=[
        pltpu.VMEM((CH,), jnp.int32),
        pltpu.VMEM((CH,), jnp.float32),
        pltpu.VMEM((RPS,), jnp.float32),
        pltpu.VMEM_SHARED((NPAD,), jnp.float32),
    ],
)
def _deg_sc(dst_hbm, out_hbm, dst_v, ones_v, zbuf_v, deg_sh):
    c = lax.axis_index("c")
    s = lax.axis_index("s")
    wid = c * NS + s

    def _fill_ones(i, _):
        ones_v[pl.ds(i * 16, 16)] = jnp.ones((16,), jnp.float32)
        return 0

    lax.fori_loop(0, CH // 16, _fill_ones, 0)

    def _fill_zero(i, _):
        zbuf_v[pl.ds(i * 16, 16)] = jnp.zeros((16,), jnp.float32)
        return 0

    lax.fori_loop(0, RPS // 16, _fill_zero, 0)
    pltpu.sync_copy(zbuf_v, deg_sh.at[pl.ds(s * RPS, RPS)])
    plsc.subcore_barrier()

    def _step(g, _):
        pltpu.sync_copy(dst_hbm.at[pl.ds(wid * EW + g * CH, CH)], dst_v)
        pltpu.sync_copy(ones_v, deg_sh.at[dst_v], add=True)
        return 0

    lax.fori_loop(0, NCH, _step, 0)
    plsc.subcore_barrier()
    pltpu.sync_copy(deg_sh.at[pl.ds(s * RPS, RPS)],
                    out_hbm.at[c, pl.ds(s * RPS, RPS)])


def _make_scatter(dk):
    @functools.partial(
        pl.kernel,
        out_type=jax.ShapeDtypeStruct((NC, NPAD, dk), jnp.float32),
        mesh=_mesh,
        compiler_params=pltpu.CompilerParams(
            use_tc_tiling_on_sc=(dk % 128 == 0)),
        scratch_types=[
            pltpu.VMEM((CH,), jnp.int32),
            pltpu.VMEM((CH,), jnp.int32),
            pltpu.VMEM((CH, dk), jnp.float32),
            pltpu.VMEM((CH, dk), jnp.float32),
            pltpu.VMEM_SHARED((NPAD, dk), jnp.float32),
            pltpu.SemaphoreType.DMA,
        ],
    )
    def _scat(y_hbm, src_hbm, dst_hbm, out_hbm,
              src_v, dst_v, rows_v, zbuf_v, acc_sh, sem):
        c = lax.axis_index("c")
        s = lax.axis_index("s")
        wid = c * NS + s

        def _zrow(i, _):
            for j in range(dk // 16):
                zbuf_v[i, pl.ds(j * 16, 16)] = jnp.zeros((16,), jnp.float32)
            return 0

        lax.fori_loop(0, CH, _zrow, 0)
        for k in range(RPS // CH):
            pltpu.sync_copy(zbuf_v, acc_sh.at[pl.ds(s * RPS + k * CH, CH)])
        plsc.subcore_barrier()

        def _step(g, _):
            base = wid * EW + g * CH
            pltpu.sync_copy(src_hbm.at[pl.ds(base, CH)], src_v)
            pltpu.sync_copy(dst_hbm.at[pl.ds(base, CH)], dst_v)
            pltpu.async_copy(y_hbm.at[src_v], rows_v, sem).wait()
            pltpu.sync_copy(rows_v, acc_sh.at[dst_v], add=True)
            return 0

        lax.fori_loop(0, NCH, _step, 0)
        plsc.subcore_barrier()
        pltpu.sync_copy(acc_sh.at[pl.ds(s * RPS, RPS)],
                        out_hbm.at[c, pl.ds(s * RPS, RPS)])

    return _scat


_scatter128 = _make_scatter(D)
_scatter64 = _make_scatter(H)


def _ka_body(deg_ref, x_ref, w1_ref, y1_ref, dinv_ref):
    deg = deg_ref[...]
    dinv = lax.rsqrt(deg[:, 0:1] + deg[:, 1:2] + 1.0)
    xw = jnp.dot(x_ref[...], w1_ref[...], preferred_element_type=jnp.float32)
    y1_ref[...] = dinv * xw
    dinv_ref[...] = dinv


def _ka(degt, x_pad, W1):
    return pl.pallas_call(
        _ka_body,
        grid=(NB,),
        in_specs=[
            pl.BlockSpec((R, 2), lambda i: (i, 0)),
            pl.BlockSpec((R, D), lambda i: (i, 0)),
            pl.BlockSpec((D, D), lambda i: (0, 0)),
        ],
        out_specs=[
            pl.BlockSpec((R, D), lambda i: (i, 0)),
            pl.BlockSpec((R, 1), lambda i: (i, 0)),
        ],
        out_shape=[
            jax.ShapeDtypeStruct((NPAD, D), jnp.float32),
            jax.ShapeDtypeStruct((NPAD, 1), jnp.float32),
        ],
    )(degt, x_pad, W1)


def _kb_body(p_ref, y1_ref, dinv_ref, b1_ref, w2_ref, y2_ref):
    p = p_ref[...]
    dinv = dinv_ref[...]
    h1 = jnp.maximum(dinv * (p[0] + p[1] + y1_ref[...]) + b1_ref[...], 0.0)
    y2_ref[...] = dinv * jnp.dot(h1, w2_ref[...],
                                 preferred_element_type=jnp.float32)


def _kb(p, y1, dinv, b1r, W2):
    return pl.pallas_call(
        _kb_body,
        grid=(NB,),
        in_specs=[
            pl.BlockSpec((NC, R, D), lambda i: (0, i, 0)),
            pl.BlockSpec((R, D), lambda i: (i, 0)),
            pl.BlockSpec((R, 1), lambda i: (i, 0)),
            pl.BlockSpec((1, D), lambda i: (0, 0)),
            pl.BlockSpec((D, H), lambda i: (0, 0)),
        ],
        out_specs=pl.BlockSpec((R, H), lambda i: (i, 0)),
        out_shape=jax.ShapeDtypeStruct((NPAD, H), jnp.float32),
    )(p, y1, dinv, b1r, W2)


def _kc_body(q_ref, y2_ref, dinv_ref, b2_ref, batch_ref, w3_ref, b3_ref,
             out_ref, psum):
    i = pl.program_id(0)

    @pl.when(i == 0)
    def _():
        psum[...] = jnp.zeros_like(psum)

    q = q_ref[...]
    dinv = dinv_ref[...]
    h2 = jnp.maximum(dinv * (q[0] + q[1] + y2_ref[...]) + b2_ref[...], 0.0)
    bb = batch_ref[...]
    gid = lax.broadcasted_iota(jnp.int32, (1, G), 1)
    m = (bb == gid).astype(jnp.float32)
    haug = jnp.concatenate([h2, jnp.ones((R, 1), jnp.float32)], axis=1)
    psum[...] += lax.dot_general(m, haug, (((0,), (0,)), ((), ())),
                                 preferred_element_type=jnp.float32)

    @pl.when(i == pl.num_programs(0) - 1)
    def _():
        ps = psum[...]
        pooled = ps[:, :H] / jnp.maximum(ps[:, H:H + 1], 1.0)
        out_ref[...] = jnp.dot(pooled, w3_ref[...],
                               preferred_element_type=jnp.float32) + b3_ref[...]


def _kc(q, y2, dinv, b2r, batch_pad, W3, b3r):
    return pl.pallas_call(
        _kc_body,
        grid=(NB,),
        in_specs=[
            pl.BlockSpec((NC, R, H), lambda i: (0, i, 0)),
            pl.BlockSpec((R, H), lambda i: (i, 0)),
            pl.BlockSpec((R, 1), lambda i: (i, 0)),
            pl.BlockSpec((1, H), lambda i: (0, 0)),
            pl.BlockSpec((R, 1), lambda i: (i, 0)),
            pl.BlockSpec((H, C), lambda i: (0, 0)),
            pl.BlockSpec((1, C), lambda i: (0, 0)),
        ],
        out_specs=pl.BlockSpec((G, C), lambda i: (0, 0)),
        out_shape=jax.ShapeDtypeStruct((G, C), jnp.float32),
        scratch_shapes=[pltpu.VMEM((G, H + 1), jnp.float32)],
    )(q, y2, dinv, b2r, batch_pad, W3, b3r)


def kernel(x, edge_index, batch, W1, b1, W2, b2, W3, b3):
    src = edge_index[0]
    dst = edge_index[1]
    x_pad = jnp.pad(x, ((0, NPAD - N), (0, 0)))
    batch_pad = jnp.pad(batch, (0, NPAD - N), constant_values=G)
    batch_pad = batch_pad.reshape(NPAD, 1)

    degp = _deg_sc(dst)
    y1, dinv = _ka(degp.T, x_pad, W1)
    p = _scatter128(y1, src, dst)
    y2 = _kb(p, y1, dinv, b1.reshape(1, D), W2)
    q = _scatter64(y2, src, dst)
    return _kc(q, y2, dinv, b2.reshape(1, H), batch_pad, W3,
               b3.reshape(1, C))
